# Initial kernel scaffold; baseline (speedup 1.0000x reference)
import jax
import jax.numpy as jnp
from jax.experimental import pallas as pl


def kernel(x, pe_table):
    raise NotImplementedError("write your pallas kernel here")


# speedup vs baseline: 1.4349x; 1.4349x over previous
import jax
import jax.numpy as jnp
from jax.experimental import pallas as pl


_BS = 512


def _add_body(x_ref, pe_ref, o_ref):
    o_ref[...] = x_ref[...] + pe_ref[...][None, :, :]


def kernel(x, pe_table):
    batch, seq, d = x.shape
    num_blocks = seq // _BS
    return pl.pallas_call(
        _add_body,
        grid=(num_blocks, batch),
        in_specs=[
            pl.BlockSpec((1, _BS, d), lambda i, j: (j, i, 0)),
            pl.BlockSpec((_BS, d), lambda i, j: (i, 0)),
        ],
        out_specs=pl.BlockSpec((1, _BS, d), lambda i, j: (j, i, 0)),
        out_shape=jax.ShapeDtypeStruct(x.shape, x.dtype),
    )(x, pe_table)
